# baseline (speedup 1.0000x reference)
import functools

import jax
import jax.numpy as jnp
from jax import lax
from jax.experimental import pallas as pl
from jax.experimental.pallas import tpu as pltpu
from jax.experimental.pallas import tpu_sc as plsc

N = 10000
E_EDGES = 160000
NG = 9
H = 64
HALF = 5120
C2 = 2000
C2P = 2048


def _mlp_body(xa_ref, xb_ref, w_ref, b_ref, g_ref, be_ref, h_ref):
    xa = xa_ref[0]
    xb = xb_ref[0]
    w = w_ref[0]
    y = jnp.dot(xa, w[:, :H].T, preferred_element_type=jnp.float32)
    y = y + jnp.dot(xb, w[:, H:].T, preferred_element_type=jnp.float32)
    y = y + b_ref[0, 0][None, :]
    mu = jnp.mean(y, axis=-1, keepdims=True)
    r = y - mu
    var = jnp.mean(r * r, axis=-1, keepdims=True)
    yn = r * lax.rsqrt(var + 1e-5)
    y = yn * g_ref[0, 0][None, :] + be_ref[0, 0][None, :]
    h_ref[0] = jnp.maximum(y, 0.0)


def _mlp_layer(xa, xb, w9, b9, g9, be9):
    return pl.pallas_call(
        _mlp_body,
        grid=(NG,),
        in_specs=[
            pl.BlockSpec((1, N, H), lambda i: (i, 0, 0)),
            pl.BlockSpec((1, N, H), lambda i: (i, 0, 0)),
            pl.BlockSpec((1, H, 2 * H), lambda i: (i, 0, 0)),
            pl.BlockSpec((1, 1, H), lambda i: (i, 0, 0)),
            pl.BlockSpec((1, 1, H), lambda i: (i, 0, 0)),
            pl.BlockSpec((1, 1, H), lambda i: (i, 0, 0)),
        ],
        out_specs=pl.BlockSpec((1, N, H), lambda i: (i, 0, 0)),
        out_shape=jax.ShapeDtypeStruct((NG, N, H), jnp.float32),
    )(xa, xb, w9, b9, g9, be9)


def _segmax_body(h_ref, pk_ref, agg_ref, pk_v, idx_v, rows_v, acc, sem):
    wid = lax.axis_index("s") * 2 + lax.axis_index("c")
    e_sh = wid >> 3
    cg = (wid >> 1) & 3
    dh = wid & 1
    dbase = dh * HALF
    shard = E_EDGES // 4
    nch = shard // C2

    def tail_body(j, _):
        idx_v[pl.ds(C2 + j * 16, 16)] = jnp.zeros((16,), jnp.int32)
        return 0
    lax.fori_loop(0, (C2P - C2) // 16, tail_body, 0)

    def graph_body(g, _):
        def zero_body(r2, _):
            acc[r2] = jnp.zeros((16,), jnp.float32)
            return 0
        lax.fori_loop(0, HALF + 1, zero_body, 0)

        gb4 = (g * N) * 4

        def chunk_body(i, _):
            pltpu.sync_copy(
                pk_ref.at[pl.ds(g * E_EDGES + e_sh * shard + i * C2, C2)],
                pk_v.at[pl.ds(0, C2)])

            def idx_body(j, _):
                vp = pk_v[pl.ds(j * 16, 16)]
                idx_v[pl.ds(j * 16, 16)] = (vp >> 14) * 4 + (gb4 + cg)
                return 0
            lax.fori_loop(0, C2 // 16, idx_body, 0)

            cps = [pltpu.async_copy(
                       h_ref.at[idx_v.at[pl.ds(k * 128, 128)]],
                       rows_v.at[pl.ds(k * 128, 128)], sem)
                   for k in range(C2P // 128)]
            for cp in cps:
                cp.wait()

            def q_body(q, _):
                vp = pk_v[pl.ds(q * 16, 16)]
                dl = (vp & 16383) - dbase
                ok = (dl >= 0) & (dl < HALF)
                dcl = jnp.where(ok, dl, HALF)
                for r in range(16):
                    d = dcl[r]
                    acc[d] = jnp.maximum(acc[d], rows_v[q * 16 + r])
                return 0
            lax.fori_loop(0, C2 // 16, q_body, 0)
            return 0

        lax.fori_loop(0, nch, chunk_body, 0)
        pltpu.sync_copy(acc.at[pl.ds(0, HALF)],
                        agg_ref.at[g, e_sh, dh, cg])
        return 0

    lax.fori_loop(0, NG, graph_body, 0)


def _segmax(h_gran, packed):
    mesh = plsc.VectorSubcoreMesh(core_axis_name="c", subcore_axis_name="s")
    f = functools.partial(
        pl.kernel,
        mesh=mesh,
        out_type=jax.ShapeDtypeStruct((NG, 4, 2, 4, HALF, 16), jnp.float32),
        compiler_params=pltpu.CompilerParams(use_tc_tiling_on_sc=False),
        scratch_types=[
            pltpu.VMEM((C2P,), jnp.int32),
            pltpu.VMEM((C2P,), jnp.int32),
            pltpu.VMEM((C2P, 16), jnp.float32),
            pltpu.VMEM((HALF + 1, 16), jnp.float32),
            pltpu.SemaphoreType.DMA,
        ],
    )(_segmax_body)
    a4 = f(h_gran, packed)
    am = jnp.max(a4, axis=1)
    am = jnp.transpose(am, (0, 1, 3, 2, 4))
    return am.reshape(NG, 2 * HALF, H)[:, :N, :]


def _red_body(h_ref, a_ref, o_ref):
    hm = jnp.max(h_ref[0], axis=0)
    am = jnp.max(a_ref[0], axis=0)
    o_ref[0] = jnp.broadcast_to(jnp.concatenate([hm, am])[None, :], (8, 2 * H))


def _reduce_feats(h, agg):
    return pl.pallas_call(
        _red_body,
        grid=(NG,),
        in_specs=[
            pl.BlockSpec((1, N, H), lambda i: (i, 0, 0)),
            pl.BlockSpec((1, N, H), lambda i: (i, 0, 0)),
        ],
        out_specs=pl.BlockSpec((1, 8, 2 * H), lambda i: (i, 0, 0)),
        out_shape=jax.ShapeDtypeStruct((NG, 8, 2 * H), jnp.float32),
    )(h, agg)


def _head_body(G_ref, Wfc_ref, wa_ref, outW_ref, o_ref):
    G = G_ref[...]
    z = jnp.dot(G, Wfc_ref[...].T, preferred_element_type=jnp.float32)
    wa = wa_ref[...]
    wa_s = wa[0, :128]
    wa_d = wa[0, 128:]
    es = jnp.sum(z * wa_s[None, :], axis=1)
    ed0 = jnp.sum(z[0] * wa_d)
    e = es + ed0
    e = jnp.where(e >= 0, e, 0.01 * e)
    row = lax.broadcasted_iota(jnp.int32, (16,), 0)
    valid = (row >= 1) & (row <= 8)
    em = jnp.max(jnp.where(valid, e, -jnp.inf))
    ex = jnp.where(valid, jnp.exp(e - em), 0.0)
    alpha = ex / jnp.sum(ex)
    gh0 = jnp.sum(alpha[:, None] * z, axis=0)
    out = jnp.dot(gh0[None, :], outW_ref[...].T,
                  preferred_element_type=jnp.float32)
    o_ref[...] = jnp.broadcast_to(out, (8, 64))


def _head(G, gat_Wfc, gat_Wattn, out_W, out_b):
    Gp = jnp.zeros((16, 2 * H), jnp.float32).at[:NG].set(G)
    wa = jnp.zeros((8, 4 * H), jnp.float32).at[0].set(gat_Wattn[0])
    oW = jnp.zeros((64, 2 * H), jnp.float32).at[:out_W.shape[0]].set(out_W)
    o = pl.pallas_call(
        _head_body,
        out_shape=jax.ShapeDtypeStruct((8, 64), jnp.float32),
    )(Gp, gat_Wfc, wa, oW)
    return o[0, :out_W.shape[0]] + out_b


def kernel(agent_feature, map_feature, a_W, a_b, a_g, a_be, m_W, m_b, m_g, m_be,
           gat_Wfc, gat_Wattn, out_W, out_b, agent_edge_index, map_edge_index):
    M = map_feature.shape[0]

    x_all = jnp.concatenate([agent_feature[None], map_feature], axis=0)
    xa = x_all[..., :H]
    xb = x_all[..., H:]

    ei_all = jnp.concatenate([agent_edge_index[None], map_edge_index], axis=0)
    packed = ((ei_all[:, 0, :] << 14) | ei_all[:, 1, :]).reshape(-1)

    def stack9(a_p, m_p):
        return jnp.concatenate(
            [a_p[None], jnp.broadcast_to(m_p[None], (M,) + m_p.shape)], axis=0)

    h = agg = None
    for l in range(3):
        w9 = stack9(a_W[l], m_W[l])
        b9 = stack9(a_b[l], m_b[l]).reshape(NG, 1, H)
        g9 = stack9(a_g[l], m_g[l]).reshape(NG, 1, H)
        be9 = stack9(a_be[l], m_be[l]).reshape(NG, 1, H)
        h = _mlp_layer(xa, xb, w9, b9, g9, be9)
        agg = _segmax(h.reshape(NG * N * 4, 16), packed)
        xa, xb = h, agg

    feat = _reduce_feats(h, agg)
    G = feat[:, 0, :]
    return _head(G, gat_Wfc, gat_Wattn, out_W, out_b)
